# baseline (speedup 1.0000x reference)
import jax
import jax.numpy as jnp
from jax.experimental import pallas as pl
from jax.experimental.pallas import tpu as pltpu

BATCH = 16384
ACC_DIM = 128
HIDDEN_DIM = 64
N_CHUNKS = 16
CHUNK_ROWS = BATCH // N_CHUNKS
TILE_ROWS = 512

_DN = (((1,), (1,)), ((), ()))


def _mlp_head_body(acc_hbm, w1_ref, b1_ref, w2_ref, b2_ref, out_ref,
                   a_vmem, sems):
    def chunk_copy(j):
        return pltpu.make_async_copy(
            acc_hbm.at[pl.ds(j * CHUNK_ROWS, CHUNK_ROWS), :],
            a_vmem.at[pl.ds(j * CHUNK_ROWS, CHUNK_ROWS), :],
            sems.at[j],
        )

    for j in range(N_CHUNKS):
        chunk_copy(j).start()

    w1 = w1_ref[...].astype(jnp.bfloat16)
    b1c = b1_ref[...].reshape(HIDDEN_DIM, 1)
    w2c = w2_ref[...].reshape(HIDDEN_DIM, 1)
    b2 = b2_ref[0, 0]

    for j in range(N_CHUNKS):
        chunk_copy(j).wait()
        for g in range(CHUNK_ROWS // TILE_ROWS):
            row0 = j * CHUNK_ROWS + g * TILE_ROWS
            a = a_vmem[pl.ds(row0, TILE_ROWS), :].astype(jnp.bfloat16)
            ht = jax.lax.dot_general(w1, a, _DN,
                                     preferred_element_type=jnp.float32)
            ht = jnp.maximum(ht + b1c, 0.0)
            red = jnp.sum(ht * w2c, axis=0, keepdims=True) + b2
            out_ref[pl.ds(row0 // 128, TILE_ROWS // 128), :] = (
                red.reshape(TILE_ROWS // 128, 128))


def kernel(acc, W1, b1, W2, b2):
    out = pl.pallas_call(
        _mlp_head_body,
        in_specs=[
            pl.BlockSpec(memory_space=pltpu.MemorySpace.HBM),
            pl.BlockSpec(memory_space=pltpu.MemorySpace.VMEM),
            pl.BlockSpec(memory_space=pltpu.MemorySpace.VMEM),
            pl.BlockSpec(memory_space=pltpu.MemorySpace.VMEM),
            pl.BlockSpec(memory_space=pltpu.MemorySpace.VMEM),
        ],
        out_specs=pl.BlockSpec(memory_space=pltpu.MemorySpace.VMEM),
        out_shape=jax.ShapeDtypeStruct((BATCH // 128, 128), jnp.float32),
        scratch_shapes=[
            pltpu.VMEM((BATCH, ACC_DIM), jnp.float32),
            pltpu.SemaphoreType.DMA((N_CHUNKS,)),
        ],
    )(acc, W1, b1.reshape(1, HIDDEN_DIM), W2, b2.reshape(1, 1))
    return out.reshape(BATCH)


# speedup vs baseline: 1.1500x; 1.1500x over previous
import jax
import jax.numpy as jnp
from jax.experimental import pallas as pl
from jax.experimental.pallas import tpu as pltpu

BATCH = 16384
ACC_DIM = 128
HIDDEN_DIM = 64
N_CHUNKS = 4
CHUNK_ROWS = BATCH // N_CHUNKS
TILE_ROWS = 512

_DN = (((1,), (1,)), ((), ()))


def _mlp_head_body(acc_hbm, w1_ref, b1_ref, w2_ref, b2_ref, out_ref,
                   a_vmem, sems):
    def chunk_copy(j):
        return pltpu.make_async_copy(
            acc_hbm.at[pl.ds(j * CHUNK_ROWS, CHUNK_ROWS), :],
            a_vmem.at[pl.ds(j * CHUNK_ROWS, CHUNK_ROWS), :],
            sems.at[j],
        )

    for j in range(N_CHUNKS):
        chunk_copy(j).start()

    w1 = w1_ref[...].astype(jnp.bfloat16)
    b1c = b1_ref[...].reshape(HIDDEN_DIM, 1)
    w2c = w2_ref[...].reshape(HIDDEN_DIM, 1)
    b2 = b2_ref[0, 0]

    for j in range(N_CHUNKS):
        chunk_copy(j).wait()
        for g in range(CHUNK_ROWS // TILE_ROWS):
            row0 = j * CHUNK_ROWS + g * TILE_ROWS
            a = a_vmem[pl.ds(row0, TILE_ROWS), :].astype(jnp.bfloat16)
            ht = jax.lax.dot_general(w1, a, _DN,
                                     preferred_element_type=jnp.float32)
            ht = jnp.maximum(ht + b1c, 0.0)
            red = jnp.sum(ht * w2c, axis=0, keepdims=True) + b2
            out_ref[pl.ds(row0 // 128, TILE_ROWS // 128), :] = (
                red.reshape(TILE_ROWS // 128, 128))


def kernel(acc, W1, b1, W2, b2):
    out = pl.pallas_call(
        _mlp_head_body,
        in_specs=[
            pl.BlockSpec(memory_space=pltpu.MemorySpace.HBM),
            pl.BlockSpec(memory_space=pltpu.MemorySpace.VMEM),
            pl.BlockSpec(memory_space=pltpu.MemorySpace.VMEM),
            pl.BlockSpec(memory_space=pltpu.MemorySpace.VMEM),
            pl.BlockSpec(memory_space=pltpu.MemorySpace.VMEM),
        ],
        out_specs=pl.BlockSpec(memory_space=pltpu.MemorySpace.VMEM),
        out_shape=jax.ShapeDtypeStruct((BATCH // 128, 128), jnp.float32),
        scratch_shapes=[
            pltpu.VMEM((BATCH, ACC_DIM), jnp.float32),
            pltpu.SemaphoreType.DMA((N_CHUNKS,)),
        ],
    )(acc, W1, b1.reshape(1, HIDDEN_DIM), W2, b2.reshape(1, 1))
    return out.reshape(BATCH)
